# baseline (speedup 1.0000x reference)
import functools

import jax
import jax.numpy as jnp
from jax import lax
from jax.experimental import pallas as pl
from jax.experimental.pallas import tpu as pltpu
from jax.experimental.pallas import tpu_sc as plsc

_INFO = plsc.get_sparse_core_info()
_NW = _INFO.num_cores * _INFO.num_subcores


def _worker_id():
    return lax.axis_index("s") * _INFO.num_cores + lax.axis_index("c")


def _relayout_table(table_t, tail_flat, num_ids, units):
    n_full = num_ids // 128
    rem = num_ids - n_full * 128
    K = 4
    n_items = n_full // K
    NB = 2
    n_t = -(-n_items // _NW)
    n_t += (-n_t) % NB
    mesh = plsc.VectorSubcoreMesh(core_axis_name="c", subcore_axis_name="s")

    @functools.partial(
        pl.kernel,
        out_type=jax.ShapeDtypeStruct((num_ids * units,), jnp.float32),
        mesh=mesh,
        scratch_types=[
            [pltpu.VMEM((units, 128 * K), jnp.float32)] * NB,
            [pltpu.VMEM((128 * K * units,), jnp.float32)] * NB,
            [pltpu.SemaphoreType.DMA] * NB,
            [pltpu.SemaphoreType.DMA] * NB,
        ],
        compiler_params=pltpu.CompilerParams(
            use_tc_tiling_on_sc=True, needs_layout_passes=False
        ),
    )
    def k1(tt_hbm, tail_hbm, out_hbm, in_v, tr_v, isems, osems):
        w = _worker_id()
        if rem:
            @pl.when(w == 0)
            def _():
                pltpu.sync_copy(
                    tail_hbm,
                    out_hbm.at[pl.ds(n_full * 128 * units, rem * units)],
                )

        def blk_of(t):
            return w + _NW * t

        def fire(t, b):
            blk = blk_of(t)

            @pl.when(blk < n_items)
            def _():
                pltpu.async_copy(
                    tt_hbm.at[:, pl.ds(blk * 128 * K, 128 * K)], in_v[b], isems[b]
                )

        uiota = units * lax.iota(jnp.int32, 16)

        def transpose(b):
            @plsc.parallel_loop(0, units, 1, unroll=8)
            def _(u):
                for c0 in range(0, 128 * K, 16):
                    vals = in_v[b][u, pl.ds(c0, 16)]
                    idx = (c0 * units + u) + uiota
                    plsc.store_scatter(tr_v[b], [idx], vals)

        def wait_out(t, b):
            blk = blk_of(t)

            @pl.when(blk < n_items)
            def _():
                pltpu.make_async_copy(
                    tr_v[b],
                    out_hbm.at[pl.ds(blk * 128 * K * units, 128 * K * units)],
                    osems[b],
                ).wait()

        def drain(t, b):
            blk = blk_of(t)

            @pl.when(blk < n_items)
            def _():
                pltpu.make_async_copy(
                    tt_hbm.at[:, pl.ds(blk * 128 * K, 128 * K)], in_v[b], isems[b]
                ).wait()
                transpose(b)
                pltpu.async_copy(
                    tr_v[b],
                    out_hbm.at[pl.ds(blk * 128 * K * units, 128 * K * units)],
                    osems[b],
                )

        for b in range(NB):
            fire(b, b)

        @pl.loop(0, n_t - NB, step=NB)
        def _(t0):
            for b in range(NB):
                @pl.when(t0 + b >= NB)
                def _():
                    wait_out(t0 + b - NB, b)
                drain(t0 + b, b)
                fire(t0 + b + NB, b)

        for b in range(NB):
            wait_out(n_t - 2 * NB + b, b)
            drain(n_t - NB + b, b)
        for b in range(NB):
            wait_out(n_t - NB + b, b)

    return k1(table_t, tail_flat)


def _gather_format(table_lin, idx2, fields, nbb, units):
    CH = 2
    n_chunks = fields * nbb // CH
    c_per_w = n_chunks // _NW
    up = units // 8
    NB = 4
    mesh = plsc.VectorSubcoreMesh(core_axis_name="c", subcore_axis_name="s")

    @functools.partial(
        pl.kernel,
        out_type=jax.ShapeDtypeStruct((fields, up, nbb, 8, 128), jnp.float32),
        mesh=mesh,
        scratch_types=[
            pltpu.VMEM((c_per_w * CH, 128), jnp.int32),
            pltpu.VMEM((NB, CH * 128, units), jnp.float32),
            pltpu.VMEM((NB, up, CH, 8, 128), jnp.float32),
            [pltpu.SemaphoreType.DMA] * NB,
            [pltpu.SemaphoreType.DMA] * NB,
        ],
        compiler_params=pltpu.CompilerParams(
            use_tc_tiling_on_sc=False, needs_layout_passes=False
        ),
    )
    def k2(tab_hbm, idx_hbm, out_hbm, idx_v, rows_v, tiles_v, gsems, osems):
        w = _worker_id()
        cbase = w * c_per_w
        ciota = lax.iota(jnp.int32, 16)
        pltpu.sync_copy(idx_hbm.at[pl.ds(cbase * CH, c_per_w * CH)], idx_v)

        def fire(j, b):
            for k in range(CH):
                pltpu.async_copy(
                    tab_hbm.at[idx_v.at[j * CH + k]],
                    rows_v.at[b].at[pl.ds(128 * k, 128)],
                    gsems[b],
                )

        def wait_out(j, b):
            cid = cbase + j
            f = (cid * CH) // nbb
            bb = cid * CH - f * nbb
            for p in range(up):
                pltpu.make_async_copy(
                    tiles_v.at[b].at[p], out_hbm.at[f, p, pl.ds(bb, CH)], osems[b]
                ).wait()

        def drain(j, b):
            for k in range(CH):
                pltpu.make_async_copy(
                    tab_hbm.at[idx_v.at[j * CH + k]],
                    rows_v.at[b].at[pl.ds(128 * k, 128)],
                    gsems[b],
                ).wait()
            cid = cbase + j
            f = (cid * CH) // nbb
            bb = cid * CH - f * nbb
            @plsc.parallel_loop(0, units, 1, unroll=4)
            def _(u):
                p = u // 8
                q = u - 8 * p
                ubc = jnp.full((16,), u, jnp.int32)
                for k in range(CH):
                    for c0 in range(0, 128, 16):
                        vals = plsc.load_gather(
                            rows_v.at[b], [128 * k + c0 + ciota, ubc]
                        )
                        tiles_v[b, p, k, q, pl.ds(c0, 16)] = vals
            for p in range(up):
                pltpu.async_copy(
                    tiles_v.at[b].at[p], out_hbm.at[f, p, pl.ds(bb, CH)], osems[b]
                )

        for b in range(NB):
            fire(b, b)

        @pl.loop(0, c_per_w - NB, step=NB)
        def _(j0):
            for b in range(NB):
                @pl.when(j0 + b >= NB)
                def _():
                    wait_out(j0 + b - NB, b)
                drain(j0 + b, b)
                fire(j0 + b + NB, b)

        for b in range(NB):
            wait_out(c_per_w - 2 * NB + b, b)
            drain(c_per_w - NB + b, b)
        for b in range(NB):
            wait_out(c_per_w - NB + b, b)

    return k2(table_lin, idx2)


@functools.partial(jax.jit, static_argnames=("num_ids", "units", "batch", "fields"))
def _embedding(inputs, table, num_ids, units, batch, fields):
    nbb = batch // 128
    n_full = num_ids // 128
    tail_flat = table[n_full * 128 :, :].reshape(-1)
    lin = _relayout_table(table.T, tail_flat, num_ids, units).reshape(num_ids, units)
    idx2 = inputs.T.reshape(fields * nbb, 128)
    out5 = _gather_format(lin, idx2, fields, nbb, units)
    return out5.transpose(2, 4, 0, 1, 3).reshape(batch, fields, units)


def kernel(inputs, kernel):
    batch, fields = inputs.shape
    num_ids, units = kernel.shape
    return _embedding(inputs, kernel, num_ids, units, batch, fields)


# speedup vs baseline: 2.1424x; 2.1424x over previous
import functools

import jax
import jax.numpy as jnp
from jax import lax
from jax.experimental import pallas as pl
from jax.experimental.pallas import tpu as pltpu
from jax.experimental.pallas import tpu_sc as plsc

_INFO = plsc.get_sparse_core_info()
_NW = _INFO.num_cores * _INFO.num_subcores


def _worker_id():
    return lax.axis_index("s") * _INFO.num_cores + lax.axis_index("c")


def _relayout_table(table_t, tail_flat, num_ids, units):
    n_full = num_ids // 128
    rem = num_ids - n_full * 128
    K = 4
    n_items = n_full // K
    NB = 2
    n_t = -(-n_items // _NW)
    n_t += (-n_t) % NB
    mesh = plsc.VectorSubcoreMesh(core_axis_name="c", subcore_axis_name="s")

    @functools.partial(
        pl.kernel,
        out_type=jax.ShapeDtypeStruct((num_ids * units,), jnp.float32),
        mesh=mesh,
        scratch_types=[
            [pltpu.VMEM((units, 128 * K), jnp.float32)] * NB,
            [pltpu.VMEM((128 * K * units,), jnp.float32)] * NB,
            [pltpu.SemaphoreType.DMA] * NB,
            [pltpu.SemaphoreType.DMA] * NB,
        ],
        compiler_params=pltpu.CompilerParams(
            use_tc_tiling_on_sc=True, needs_layout_passes=False
        ),
    )
    def k1(tt_hbm, tail_hbm, out_hbm, in_v, tr_v, isems, osems):
        w = _worker_id()
        if rem:
            @pl.when(w == 0)
            def _():
                pltpu.sync_copy(
                    tail_hbm,
                    out_hbm.at[pl.ds(n_full * 128 * units, rem * units)],
                )

        def blk_of(t):
            return w + _NW * t

        def fire(t, b):
            blk = blk_of(t)

            @pl.when(blk < n_items)
            def _():
                pltpu.async_copy(
                    tt_hbm.at[:, pl.ds(blk * 128 * K, 128 * K)], in_v[b], isems[b]
                )

        iot = lax.iota(jnp.int32, 16)
        W = 128 * K

        def transpose(b):
            for j in range(16):
                rot = (iot + j) & 15
                spat = rot * units + iot

                @plsc.parallel_loop(0, W // 16, 1, unroll=4)
                def _(ci):
                    c0 = 16 * ci
                    for u0 in range(0, units, 16):
                        vals = plsc.load_gather(in_v[b], [u0 + iot, c0 + rot])
                        plsc.store_scatter(
                            tr_v[b], [c0 * units + u0 + spat], vals
                        )

        def wait_out(t, b):
            blk = blk_of(t)

            @pl.when(blk < n_items)
            def _():
                pltpu.make_async_copy(
                    tr_v[b],
                    out_hbm.at[pl.ds(blk * 128 * K * units, 128 * K * units)],
                    osems[b],
                ).wait()

        def drain(t, b):
            blk = blk_of(t)

            @pl.when(blk < n_items)
            def _():
                pltpu.make_async_copy(
                    tt_hbm.at[:, pl.ds(blk * 128 * K, 128 * K)], in_v[b], isems[b]
                ).wait()
                transpose(b)
                pltpu.async_copy(
                    tr_v[b],
                    out_hbm.at[pl.ds(blk * 128 * K * units, 128 * K * units)],
                    osems[b],
                )

        for b in range(NB):
            fire(b, b)

        @pl.loop(0, n_t - NB, step=NB)
        def _(t0):
            for b in range(NB):
                @pl.when(t0 + b >= NB)
                def _():
                    wait_out(t0 + b - NB, b)
                drain(t0 + b, b)
                fire(t0 + b + NB, b)

        for b in range(NB):
            wait_out(n_t - 2 * NB + b, b)
            drain(n_t - NB + b, b)
        for b in range(NB):
            wait_out(n_t - NB + b, b)

    return k1(table_t, tail_flat)


def _gather_format(table_lin, idx2, fields, nbb, units):
    CH = 2
    n_chunks = fields * nbb // CH
    c_per_w = n_chunks // _NW
    up = units // 8
    NB = 4
    mesh = plsc.VectorSubcoreMesh(core_axis_name="c", subcore_axis_name="s")

    @functools.partial(
        pl.kernel,
        out_type=jax.ShapeDtypeStruct((fields, up, nbb, 8, 128), jnp.float32),
        mesh=mesh,
        scratch_types=[
            pltpu.VMEM((c_per_w * CH, 128), jnp.int32),
            pltpu.VMEM((NB, CH * 128, units), jnp.float32),
            pltpu.VMEM((NB, up, CH, 8, 128), jnp.float32),
            [pltpu.SemaphoreType.DMA] * NB,
            [pltpu.SemaphoreType.DMA] * NB,
        ],
        compiler_params=pltpu.CompilerParams(
            use_tc_tiling_on_sc=False, needs_layout_passes=False
        ),
    )
    def k2(tab_hbm, idx_hbm, out_hbm, idx_v, rows_v, tiles_v, gsems, osems):
        w = _worker_id()
        cbase = w * c_per_w
        ciota = lax.iota(jnp.int32, 16)
        pltpu.sync_copy(idx_hbm.at[pl.ds(cbase * CH, c_per_w * CH)], idx_v)

        def fire(j, b):
            for k in range(CH):
                pltpu.async_copy(
                    tab_hbm.at[idx_v.at[j * CH + k]],
                    rows_v.at[b].at[pl.ds(128 * k, 128)],
                    gsems[b],
                )

        def wait_out(j, b):
            cid = cbase + j
            f = (cid * CH) // nbb
            bb = cid * CH - f * nbb
            for p in range(up):
                pltpu.make_async_copy(
                    tiles_v.at[b].at[p], out_hbm.at[f, p, pl.ds(bb, CH)], osems[b]
                ).wait()

        def drain(j, b):
            for k in range(CH):
                pltpu.make_async_copy(
                    tab_hbm.at[idx_v.at[j * CH + k]],
                    rows_v.at[b].at[pl.ds(128 * k, 128)],
                    gsems[b],
                ).wait()
            cid = cbase + j
            f = (cid * CH) // nbb
            bb = cid * CH - f * nbb
            @plsc.parallel_loop(0, units, 1, unroll=4)
            def _(u):
                p = u // 8
                q = u - 8 * p
                ubc = jnp.full((16,), u, jnp.int32)
                for k in range(CH):
                    for c0 in range(0, 128, 16):
                        vals = plsc.load_gather(
                            rows_v.at[b], [128 * k + c0 + ciota, ubc]
                        )
                        tiles_v[b, p, k, q, pl.ds(c0, 16)] = vals
            for p in range(up):
                pltpu.async_copy(
                    tiles_v.at[b].at[p], out_hbm.at[f, p, pl.ds(bb, CH)], osems[b]
                )

        for b in range(NB):
            fire(b, b)

        @pl.loop(0, c_per_w - NB, step=NB)
        def _(j0):
            for b in range(NB):
                @pl.when(j0 + b >= NB)
                def _():
                    wait_out(j0 + b - NB, b)
                drain(j0 + b, b)
                fire(j0 + b + NB, b)

        for b in range(NB):
            wait_out(c_per_w - 2 * NB + b, b)
            drain(c_per_w - NB + b, b)
        for b in range(NB):
            wait_out(c_per_w - NB + b, b)

    return k2(table_lin, idx2)


@functools.partial(jax.jit, static_argnames=("num_ids", "units", "batch", "fields"))
def _embedding(inputs, table, num_ids, units, batch, fields):
    nbb = batch // 128
    n_full = num_ids // 128
    tail_flat = table[n_full * 128 :, :].reshape(-1)
    lin = _relayout_table(table.T, tail_flat, num_ids, units).reshape(num_ids, units)
    idx2 = inputs.T.reshape(fields * nbb, 128)
    out5 = _gather_format(lin, idx2, fields, nbb, units)
    return out5.transpose(2, 4, 0, 1, 3).reshape(batch, fields, units)


def kernel(inputs, kernel):
    batch, fields = inputs.shape
    num_ids, units = kernel.shape
    return _embedding(inputs, kernel, num_ids, units, batch, fields)


# speedup vs baseline: 3.2855x; 1.5336x over previous
import functools

import jax
import jax.numpy as jnp
from jax import lax
from jax.experimental import pallas as pl
from jax.experimental.pallas import tpu as pltpu
from jax.experimental.pallas import tpu_sc as plsc

_INFO = plsc.get_sparse_core_info()
_NW = _INFO.num_cores * _INFO.num_subcores


def _worker_id():
    return lax.axis_index("s") * _INFO.num_cores + lax.axis_index("c")


def _relayout_table(table_t, tail_flat, num_ids, units):
    n_full = num_ids // 128
    rem = num_ids - n_full * 128
    K = 4
    n_items = n_full // K
    NB = 2
    n_t = -(-n_items // _NW)
    n_t += (-n_t) % NB
    mesh = plsc.VectorSubcoreMesh(core_axis_name="c", subcore_axis_name="s")

    @functools.partial(
        pl.kernel,
        out_type=jax.ShapeDtypeStruct((num_ids * units,), jnp.float32),
        mesh=mesh,
        scratch_types=[
            [pltpu.VMEM((units, 128 * K), jnp.float32)] * NB,
            [pltpu.VMEM((128 * K * units,), jnp.float32)] * NB,
            [pltpu.SemaphoreType.DMA] * NB,
            [pltpu.SemaphoreType.DMA] * NB,
        ],
        compiler_params=pltpu.CompilerParams(
            use_tc_tiling_on_sc=True, needs_layout_passes=False
        ),
    )
    def k1(tt_hbm, tail_hbm, out_hbm, in_v, tr_v, isems, osems):
        w = _worker_id()
        if rem:
            @pl.when(w == 0)
            def _():
                pltpu.sync_copy(
                    tail_hbm,
                    out_hbm.at[pl.ds(n_full * 128 * units, rem * units)],
                )

        def blk_of(t):
            return w + _NW * t

        def fire(t, b):
            blk = blk_of(t)

            @pl.when(blk < n_items)
            def _():
                pltpu.async_copy(
                    tt_hbm.at[:, pl.ds(blk * 128 * K, 128 * K)], in_v[b], isems[b]
                )

        iot = lax.iota(jnp.int32, 16)
        W = 128 * K

        def transpose(b):
            for j in range(16):
                rot = (iot + j) & 15
                spat = rot * units + iot

                @plsc.parallel_loop(0, W // 16, 1, unroll=4)
                def _(ci):
                    c0 = 16 * ci
                    for u0 in range(0, units, 16):
                        vals = plsc.load_gather(in_v[b], [u0 + iot, c0 + rot])
                        plsc.store_scatter(
                            tr_v[b], [c0 * units + u0 + spat], vals
                        )

        def wait_out(t, b):
            blk = blk_of(t)

            @pl.when(blk < n_items)
            def _():
                pltpu.make_async_copy(
                    tr_v[b],
                    out_hbm.at[pl.ds(blk * 128 * K * units, 128 * K * units)],
                    osems[b],
                ).wait()

        def drain(t, b):
            blk = blk_of(t)

            @pl.when(blk < n_items)
            def _():
                pltpu.make_async_copy(
                    tt_hbm.at[:, pl.ds(blk * 128 * K, 128 * K)], in_v[b], isems[b]
                ).wait()
                transpose(b)
                pltpu.async_copy(
                    tr_v[b],
                    out_hbm.at[pl.ds(blk * 128 * K * units, 128 * K * units)],
                    osems[b],
                )

        for b in range(NB):
            fire(b, b)

        @pl.loop(0, n_t - NB, step=NB)
        def _(t0):
            for b in range(NB):
                @pl.when(t0 + b >= NB)
                def _():
                    wait_out(t0 + b - NB, b)
                drain(t0 + b, b)
                fire(t0 + b + NB, b)

        for b in range(NB):
            wait_out(n_t - 2 * NB + b, b)
            drain(n_t - NB + b, b)
        for b in range(NB):
            wait_out(n_t - NB + b, b)

    return k1(table_t, tail_flat)


def _gather_format(table_lin, idx2, fields, nbb, units):
    CH = 2
    n_chunks = fields * nbb // CH
    c_per_w = n_chunks // _NW
    up = units // 8
    NB = 4
    mesh = plsc.VectorSubcoreMesh(core_axis_name="c", subcore_axis_name="s")

    @functools.partial(
        pl.kernel,
        out_type=jax.ShapeDtypeStruct((fields * up * nbb * 8 * 128,), jnp.float32),
        mesh=mesh,
        scratch_types=[
            pltpu.VMEM((c_per_w * CH, 128), jnp.int32),
            pltpu.VMEM((NB, CH * 128, units), jnp.float32),
            pltpu.VMEM((NB, up * CH * 8 * 128), jnp.float32),
            [pltpu.SemaphoreType.DMA] * NB,
            [pltpu.SemaphoreType.DMA] * NB,
        ],
        compiler_params=pltpu.CompilerParams(
            use_tc_tiling_on_sc=False, needs_layout_passes=False
        ),
    )
    def k2(tab_hbm, idx_hbm, out_hbm, idx_v, rows_v, tiles_v, gsems, osems):
        w = _worker_id()
        cbase = w * c_per_w
        ciota = lax.iota(jnp.int32, 16)
        pltpu.sync_copy(idx_hbm.at[pl.ds(cbase * CH, c_per_w * CH)], idx_v)

        def fire(j, b):
            for k in range(CH):
                pltpu.async_copy(
                    tab_hbm.at[idx_v.at[j * CH + k]],
                    rows_v.at[b].at[pl.ds(128 * k, 128)],
                    gsems[b],
                )

        def wait_out(j, b):
            cid = cbase + j
            f = (cid * CH) // nbb
            bb = cid * CH - f * nbb
            for p in range(up):
                pltpu.make_async_copy(
                    tiles_v.at[b].at[pl.ds(p * CH * 1024, CH * 1024)],
                    out_hbm.at[pl.ds(((f * up + p) * nbb + bb) * 1024, CH * 1024)],
                    osems[b],
                ).wait()

        def drain(j, b):
            for k in range(CH):
                pltpu.make_async_copy(
                    tab_hbm.at[idx_v.at[j * CH + k]],
                    rows_v.at[b].at[pl.ds(128 * k, 128)],
                    gsems[b],
                ).wait()
            cid = cbase + j
            f = (cid * CH) // nbb
            bb = cid * CH - f * nbb
            for dj in range(16):
                rot = (ciota + dj) & 15

                @plsc.parallel_loop(0, CH * 8, 1, unroll=4)
                def _(ci):
                    k = ci // 8
                    cc0 = 16 * (ci - 8 * k)
                    for u0 in range(0, units, 16):
                        pat = (
                            ((u0 + rot) // 8) * (CH * 1024)
                            + ((u0 + rot) & 7) * 128
                            + ciota
                        )
                        vals = plsc.load_gather(
                            rows_v.at[b], [128 * k + cc0 + ciota, u0 + rot]
                        )
                        plsc.store_scatter(
                            tiles_v.at[b], [k * 1024 + cc0 + pat], vals
                        )
            for p in range(up):
                pltpu.async_copy(
                    tiles_v.at[b].at[pl.ds(p * CH * 1024, CH * 1024)],
                    out_hbm.at[pl.ds(((f * up + p) * nbb + bb) * 1024, CH * 1024)],
                    osems[b],
                )

        for b in range(NB):
            fire(b, b)

        @pl.loop(0, c_per_w - NB, step=NB)
        def _(j0):
            for b in range(NB):
                @pl.when(j0 + b >= NB)
                def _():
                    wait_out(j0 + b - NB, b)
                drain(j0 + b, b)
                fire(j0 + b + NB, b)

        for b in range(NB):
            wait_out(c_per_w - 2 * NB + b, b)
            drain(c_per_w - NB + b, b)
        for b in range(NB):
            wait_out(c_per_w - NB + b, b)

    return k2(table_lin, idx2)


@functools.partial(jax.jit, static_argnames=("num_ids", "units", "batch", "fields"))
def _embedding(inputs, table, num_ids, units, batch, fields):
    nbb = batch // 128
    n_full = num_ids // 128
    tail_flat = table[n_full * 128 :, :].reshape(-1)
    lin = _relayout_table(table.T, tail_flat, num_ids, units).reshape(num_ids, units)
    idx2 = inputs.T.reshape(fields * nbb, 128)
    out5 = _gather_format(lin, idx2, fields, nbb, units).reshape(
        fields, units // 8, nbb, 8, 128
    )
    return out5.transpose(2, 4, 0, 1, 3).reshape(batch, fields, units)


def kernel(inputs, kernel):
    batch, fields = inputs.shape
    num_ids, units = kernel.shape
    return _embedding(inputs, kernel, num_ids, units, batch, fields)
